# baseline (speedup 1.0000x reference)
import functools

import jax
import jax.numpy as jnp
from jax import lax
from jax.experimental import pallas as pl
from jax.experimental.pallas import tpu as pltpu
from jax.experimental.pallas import tpu_sc as plsc


def _make_sc_gather(V, D, B):
  info = plsc.get_sparse_core_info()
  NC, NS = info.num_cores, info.num_subcores
  NW = NC * NS
  assert B % (8 * NW) == 0
  b_per_w = B // NW
  mesh = plsc.VectorSubcoreMesh(core_axis_name="c", subcore_axis_name="s")

  @functools.partial(
      pl.kernel,
      mesh=mesh,
      out_type=jax.ShapeDtypeStruct((B, D), jnp.float32),
      compiler_params=pltpu.CompilerParams(use_tc_tiling_on_sc=False),
      scratch_types=[
          pltpu.VMEM((b_per_w,), jnp.int32),
          pltpu.VMEM((b_per_w, D), jnp.float32),
          pltpu.SemaphoreType.DMA,
      ],
  )
  def gather_k(table_hbm, idx_hbm, out_hbm, idx_v, rows_v, sem):
    wid = lax.axis_index("s") * NC + lax.axis_index("c")
    base = wid * b_per_w
    pltpu.sync_copy(idx_hbm.at[pl.ds(base, b_per_w)], idx_v)
    pltpu.async_copy(table_hbm.at[idx_v], rows_v, sem).wait()
    pltpu.sync_copy(rows_v, out_hbm.at[pl.ds(base, b_per_w)])

  return gather_k


def _softmax_matmul_body(g_ref, p_ref, o_ref):
  w = g_ref[...]
  m = jnp.max(w, axis=-1, keepdims=True)
  e = jnp.exp(w - m)
  s = jnp.sum(e, axis=-1, keepdims=True)
  o_ref[...] = jnp.dot(e / s, p_ref[...], preferred_element_type=jnp.float32)


def _softmax_matmul(g, P):
  B, K = g.shape
  D = P.shape[1]
  BLK = 2048
  return pl.pallas_call(
      _softmax_matmul_body,
      grid=(B // BLK,),
      in_specs=[
          pl.BlockSpec((BLK, K), lambda i: (i, 0)),
          pl.BlockSpec((K, D), lambda i: (0, 0)),
      ],
      out_specs=pl.BlockSpec((BLK, D), lambda i: (i, 0)),
      out_shape=jax.ShapeDtypeStruct((B, D), jnp.float32),
  )(g, P)


def kernel(W, P, u_ids):
  V, K = W.shape
  B = u_ids.shape[0]
  g = _make_sc_gather(V, K, B)(W, u_ids.astype(jnp.int32))
  return _softmax_matmul(g, P)
